# Initial kernel scaffold; baseline (speedup 1.0000x reference)
import jax
import jax.numpy as jnp
from jax.experimental import pallas as pl


def kernel(x, edge_index, edge_weight, params):
    raise NotImplementedError("write your pallas kernel here")


# speedup vs baseline: 1.0653x; 1.0653x over previous
import math
import jax
import jax.numpy as jnp
from jax.experimental import pallas as pl
from jax.experimental.pallas import tpu as pltpu

C = 128
DEPTH = 3
RATIO = 0.5
_ISQ = 1.0 / math.sqrt(float(C))


def _proj_body(x_ref, w_ref, b_ref, y_ref):
    y_ref[...] = (
        jnp.dot(x_ref[...], w_ref[...], preferred_element_type=jnp.float32)
        + b_ref[...]
    )


def _project(xp, Wcat, bcat):
    P = xp.shape[0]
    BR = 256
    return pl.pallas_call(
        _proj_body,
        grid=(P // BR,),
        in_specs=[
            pl.BlockSpec((BR, C), lambda i: (i, 0)),
            pl.BlockSpec((C, 5 * C), lambda i: (0, 0)),
            pl.BlockSpec((5 * C,), lambda i: (0,)),
        ],
        out_specs=pl.BlockSpec((BR, 5 * C), lambda i: (i, 0)),
        out_shape=jax.ShapeDtypeStruct((P, 5 * C), jnp.float32),
    )(xp, Wcat, bcat)


def _prep_params(p):
    we = p["We"][0]
    Wcat = jnp.concatenate(
        [p["Wq"], p["Wk"], p["Wv"], p["Ws"],
         jnp.zeros((C, C), jnp.float32).at[:, 0].set(p["Wq"] @ we)],
        axis=1,
    )
    bcat = jnp.concatenate(
        [p["bq"], p["bk"], p["bv"], p["bs"],
         jnp.zeros((C,), jnp.float32).at[0].set(p["bq"] @ we)],
    )
    wb = p["Wb"][:, 0]
    wb_out = wb[:C] + wb[2 * C:]
    wb_xr = wb[C:2 * C] - wb[2 * C:]
    return {"Wcat": Wcat, "bcat": bcat, "we": we,
            "wb_out": wb_out, "wb_xr": wb_xr}


def _tconv(fp, x, src, dst, ea1, valid, n):
    P = ((n + 255) // 256) * 256
    xp = jnp.zeros((P, C), jnp.float32).at[:n].set(x)
    y = _project(xp, fp["Wcat"], fp["bcat"])
    q = y[:n, :C]
    k = y[:n, C:2 * C]
    v = y[:n, 2 * C:3 * C]
    xr = y[:n, 3 * C:4 * C]
    qw = y[:n, 4 * C]

    vf = valid.astype(jnp.float32)
    a = (jnp.sum(q[dst] * k[src], axis=-1) + ea1 * qw[dst]) * _ISQ
    a = jnp.where(valid, a, -1e30)
    m = jax.ops.segment_max(a, dst, num_segments=n)
    m = jnp.where(jnp.isfinite(m), m, 0.0)
    pe = jnp.exp(a - m[dst]) * vf
    s = jax.ops.segment_sum(pe, dst, num_segments=n)
    w = jax.ops.segment_sum(pe * ea1, dst, num_segments=n)
    acc = jax.ops.segment_sum(pe[:, None] * v[src], dst, num_segments=n)
    out = (acc + w[:, None] * fp["we"][None, :]) / (s[:, None] + 1e-16)

    b = jax.nn.sigmoid(out @ fp["wb_out"] + xr @ fp["wb_xr"])[:, None]
    return b * xr + (1.0 - b) * out


def _pool(w, x, src, dst, ea1, valid, n):
    score = jnp.tanh((x @ w) / (jnp.linalg.norm(w) + 1e-16))
    kk = int(math.ceil(RATIO * n))
    top, perm = jax.lax.top_k(score, kk)
    xn = x[perm] * top[:, None]
    mapping = jnp.full((n,), -1, jnp.int32).at[perm].set(
        jnp.arange(kk, dtype=jnp.int32))
    ns = mapping[src]
    nd = mapping[dst]
    nv = valid & (ns >= 0) & (nd >= 0)
    ns = jnp.where(nv, ns, 0)
    nd = jnp.where(nv, nd, 0)
    return xn, ns, nd, nv, perm, kk


def kernel(x, edge_index, edge_weight, params):
    src = edge_index[0].astype(jnp.int32)
    dst = edge_index[1].astype(jnp.int32)
    ea1 = edge_weight[:, 0]
    n = x.shape[0]
    valid = jnp.ones((src.shape[0],), dtype=bool)

    fps = {name: _prep_params(params[name])
           for name in ("down_in_hid", "down_hid", "up_in_hid", "up_in_out")}

    x = jax.nn.relu(_tconv(fps["down_in_hid"], x, src, dst, ea1, valid, n))
    xs = [x]
    levels = [(src, dst, valid, n)]
    perms = []
    for i in range(DEPTH):
        x, src, dst, valid, perm, n = _pool(
            params["pool_w"][i], x, src, dst, ea1, valid, n)
        x = jax.nn.relu(_tconv(fps["down_hid"], x, src, dst, ea1, valid, n))
        if i < DEPTH - 1:
            xs.append(x)
            levels.append((src, dst, valid, n))
        perms.append(perm)
    for i in range(DEPTH):
        j = DEPTH - 1 - i
        res = xs[j]
        src, dst, valid, n = levels[j]
        perm = perms[j]
        up = jnp.zeros_like(res).at[perm].set(x)
        x = res + up
        fp = fps["up_in_hid"] if i < DEPTH - 1 else fps["up_in_out"]
        x = _tconv(fp, x, src, dst, ea1, valid, n)
        if i < DEPTH - 1:
            x = jax.nn.relu(x)
    return x
